# baseline (device time: 24157 ns/iter reference)
import functools

import jax
import jax.numpy as jnp
from jax import lax
from jax.experimental import pallas as pl
from jax.experimental.pallas import tpu as pltpu

N_DEV = 4
HALO = 128


def kernel(x, Wq, K_ext, V_ext, Wo):
    B, Sl, D = x.shape
    _, _, H, Dh = K_ext.shape
    Dq = Wq.shape[1]
    Sf = Sl + 2 * HALO

    Kt = jnp.transpose(K_ext, (0, 2, 1, 3))
    Vt = jnp.transpose(V_ext, (0, 2, 1, 3))

    def body(x_ref, wq_ref, k_ref, v_ref, wo_ref, out_ref,
             kf_ref, vf_ref, q_ref, send_sems, recv_sems):
        my = lax.axis_index("i")
        has_left = my > 0
        has_right = my < N_DEV - 1

        def mk(src, dst, s_idx, r_idx, dev):
            return pltpu.make_async_remote_copy(
                src_ref=src,
                dst_ref=dst,
                send_sem=send_sems.at[s_idx],
                recv_sem=recv_sems.at[r_idx],
                device_id=(dev,),
                device_id_type=pl.DeviceIdType.MESH,
            )

        def left_halo(ref):
            return ref.at[:, :, pl.ds(0, HALO), :]

        def right_halo(ref):
            return ref.at[:, :, pl.ds(HALO + Sl, HALO), :]

        def own_left_edge(ref):
            return ref.at[:, :, pl.ds(0, HALO), :]

        def own_right_edge(ref):
            return ref.at[:, :, pl.ds(Sl - HALO, HALO), :]

        barrier_sem = pltpu.get_barrier_semaphore()
        sig_l = jnp.where(has_left, my - 1, my)
        sig_r = jnp.where(has_right, my + 1, my)
        for tgt in (sig_l, sig_r):
            pl.semaphore_signal(
                barrier_sem, inc=1,
                device_id=(tgt,), device_id_type=pl.DeviceIdType.MESH,
            )
        pl.semaphore_wait(barrier_sem, 2)

        @pl.when(has_right)
        def _():
            mk(own_right_edge(k_ref), left_halo(kf_ref), 0, 0, my + 1).start()
            mk(own_right_edge(v_ref), left_halo(vf_ref), 1, 1, my + 1).start()

        @pl.when(has_left)
        def _():
            mk(own_left_edge(k_ref), right_halo(kf_ref), 2, 2, my - 1).start()
            mk(own_left_edge(v_ref), right_halo(vf_ref), 3, 3, my - 1).start()

        kf_ref[:, :, pl.ds(HALO, Sl), :] = k_ref[:, :, :, :]
        vf_ref[:, :, pl.ds(HALO, Sl), :] = v_ref[:, :, :, :]
        for b in range(B):
            q_ref[b] = jnp.dot(
                x_ref[b], wq_ref[:, :], preferred_element_type=jnp.float32
            )

        @pl.when(has_left)
        def _():
            mk(own_left_edge(k_ref), left_halo(kf_ref), 0, 0, my).wait_recv()
            mk(own_left_edge(v_ref), left_halo(vf_ref), 1, 1, my).wait_recv()

        @pl.when(has_right)
        def _():
            mk(own_left_edge(k_ref), right_halo(kf_ref), 2, 2, my).wait_recv()
            mk(own_left_edge(v_ref), right_halo(vf_ref), 3, 3, my).wait_recv()

        i_idx = lax.broadcasted_iota(jnp.int32, (Sl, Sf), 0)
        j_idx = lax.broadcasted_iota(jnp.int32, (Sl, Sf), 1)
        kg = my * Sl - HALO + j_idx
        mask = (
            (j_idx >= i_idx)
            & (j_idx <= i_idx + 2 * HALO)
            & (kg >= 0)
            & (kg < N_DEV * Sl)
        )

        for b in range(B):
            acc = jnp.zeros((Sl, D), jnp.float32)
            for h in range(H):
                q_bh = q_ref[b, :, pl.ds(h * Dh, Dh)]
                k_bh = kf_ref[b, h]
                s = lax.dot_general(
                    q_bh, k_bh, (((1,), (1,)), ((), ())),
                    preferred_element_type=jnp.float32,
                ) * 0.125
                s = jnp.where(mask, s, -1e9)
                m = jnp.max(s, axis=1, keepdims=True)
                w = jnp.exp(s - m)
                w = w / jnp.sum(w, axis=1, keepdims=True)
                ctx = jnp.dot(
                    w, vf_ref[b, h], preferred_element_type=jnp.float32
                )
                acc = acc + jnp.dot(
                    ctx, wo_ref[pl.ds(h * Dh, Dh), :],
                    preferred_element_type=jnp.float32,
                )
            out_ref[b] = acc

        @pl.when(has_right)
        def _():
            mk(own_right_edge(k_ref), left_halo(kf_ref), 0, 0, my).wait_send()
            mk(own_right_edge(v_ref), left_halo(vf_ref), 1, 1, my).wait_send()

        @pl.when(has_left)
        def _():
            mk(own_left_edge(k_ref), right_halo(kf_ref), 2, 2, my).wait_send()
            mk(own_left_edge(v_ref), right_halo(vf_ref), 3, 3, my).wait_send()

    return pl.pallas_call(
        body,
        out_shape=jax.ShapeDtypeStruct((B, Sl, D), jnp.float32),
        in_specs=[pl.BlockSpec(memory_space=pltpu.VMEM)] * 5,
        out_specs=pl.BlockSpec(memory_space=pltpu.VMEM),
        scratch_shapes=[
            pltpu.VMEM((B, H, Sf, Dh), jnp.float32),
            pltpu.VMEM((B, H, Sf, Dh), jnp.float32),
            pltpu.VMEM((B, Sl, Dq), jnp.float32),
            pltpu.SemaphoreType.DMA((4,)),
            pltpu.SemaphoreType.DMA((4,)),
        ],
        compiler_params=pltpu.CompilerParams(collective_id=0),
    )(x, Wq, Kt, Vt, Wo)


# device time: 22777 ns/iter; 1.0606x vs baseline; 1.0606x over previous
import functools

import jax
import jax.numpy as jnp
from jax import lax
from jax.experimental import pallas as pl
from jax.experimental.pallas import tpu as pltpu

N_DEV = 4
HALO = 128


def kernel(x, Wq, K_ext, V_ext, Wo):
    B, Sl, D = x.shape
    _, _, H, Dh = K_ext.shape
    Dq = Wq.shape[1]
    Sf = Sl + 2 * HALO

    Kt = jnp.transpose(K_ext, (0, 2, 1, 3))
    Vt = jnp.transpose(V_ext, (0, 2, 1, 3))

    def body(x_ref, wq_ref, k_ref, v_ref, wo_ref, out_ref,
             kf_ref, vf_ref, q_ref, send_sems, recv_sems):
        my = lax.axis_index("i")
        has_left = my > 0
        has_right = my < N_DEV - 1

        def mk(src, dst, s_idx, r_idx, dev):
            return pltpu.make_async_remote_copy(
                src_ref=src,
                dst_ref=dst,
                send_sem=send_sems.at[s_idx],
                recv_sem=recv_sems.at[r_idx],
                device_id=(dev,),
                device_id_type=pl.DeviceIdType.MESH,
            )

        def left_halo(ref):
            return ref.at[:, :, pl.ds(0, HALO), :]

        def right_halo(ref):
            return ref.at[:, :, pl.ds(HALO + Sl, HALO), :]

        def own_left_edge(ref):
            return ref.at[:, :, pl.ds(0, HALO), :]

        def own_right_edge(ref):
            return ref.at[:, :, pl.ds(Sl - HALO, HALO), :]

        barrier_sem = pltpu.get_barrier_semaphore()
        sig_l = jnp.where(has_left, my - 1, my)
        sig_r = jnp.where(has_right, my + 1, my)
        for tgt in (sig_l, sig_r):
            pl.semaphore_signal(
                barrier_sem, inc=1,
                device_id=(tgt,), device_id_type=pl.DeviceIdType.MESH,
            )
        pl.semaphore_wait(barrier_sem, 2)

        @pl.when(has_right)
        def _():
            mk(own_right_edge(k_ref), left_halo(kf_ref), 0, 0, my + 1).start()
            mk(own_right_edge(v_ref), left_halo(vf_ref), 1, 1, my + 1).start()

        @pl.when(has_left)
        def _():
            mk(own_left_edge(k_ref), right_halo(kf_ref), 2, 2, my - 1).start()
            mk(own_left_edge(v_ref), right_halo(vf_ref), 3, 3, my - 1).start()

        kf_ref[:, :, pl.ds(HALO, Sl), :] = k_ref[:, :, :, :]
        vf_ref[:, :, pl.ds(HALO, Sl), :] = v_ref[:, :, :, :]

        @pl.when(jnp.logical_not(has_left))
        def _():
            kf_ref[:, :, pl.ds(0, HALO), :] = jnp.zeros(
                (B, H, HALO, Dh), jnp.float32
            )
            vf_ref[:, :, pl.ds(0, HALO), :] = jnp.zeros(
                (B, H, HALO, Dh), jnp.float32
            )

        @pl.when(jnp.logical_not(has_right))
        def _():
            kf_ref[:, :, pl.ds(HALO + Sl, HALO), :] = jnp.zeros(
                (B, H, HALO, Dh), jnp.float32
            )
            vf_ref[:, :, pl.ds(HALO + Sl, HALO), :] = jnp.zeros(
                (B, H, HALO, Dh), jnp.float32
            )

        for b in range(B):
            q_ref[b] = jnp.dot(
                x_ref[b], wq_ref[:, :], preferred_element_type=jnp.float32
            ) * 0.125

        @pl.when(has_left)
        def _():
            mk(own_left_edge(k_ref), left_halo(kf_ref), 0, 0, my).wait_recv()
            mk(own_left_edge(v_ref), left_halo(vf_ref), 1, 1, my).wait_recv()

        @pl.when(has_right)
        def _():
            mk(own_left_edge(k_ref), right_halo(kf_ref), 2, 2, my).wait_recv()
            mk(own_left_edge(v_ref), right_halo(vf_ref), 3, 3, my).wait_recv()

        i_idx = lax.broadcasted_iota(jnp.int32, (Sl, Sf), 0)
        j_idx = lax.broadcasted_iota(jnp.int32, (Sl, Sf), 1)
        kg = my * Sl - HALO + j_idx
        maskf = (
            (j_idx >= i_idx)
            & (j_idx <= i_idx + 2 * HALO)
            & (kg >= 0)
            & (kg < N_DEV * Sl)
        ).astype(jnp.float32)

        for b in range(B):
            acc = jnp.zeros((Sl, D), jnp.float32)
            for h in range(H):
                q_bh = q_ref[b, :, pl.ds(h * Dh, Dh)]
                k_bh = kf_ref[b, h]
                s = lax.dot_general(
                    q_bh, k_bh, (((1,), (1,)), ((), ())),
                    preferred_element_type=jnp.float32,
                )
                w = jnp.exp(s) * maskf
                w_sum = jnp.sum(w, axis=1, keepdims=True)
                ctx = jnp.dot(
                    w, vf_ref[b, h], preferred_element_type=jnp.float32
                ) / w_sum
                acc = acc + jnp.dot(
                    ctx, wo_ref[pl.ds(h * Dh, Dh), :],
                    preferred_element_type=jnp.float32,
                )
            out_ref[b] = acc

        @pl.when(has_right)
        def _():
            mk(own_right_edge(k_ref), left_halo(kf_ref), 0, 0, my).wait_send()
            mk(own_right_edge(v_ref), left_halo(vf_ref), 1, 1, my).wait_send()

        @pl.when(has_left)
        def _():
            mk(own_left_edge(k_ref), right_halo(kf_ref), 2, 2, my).wait_send()
            mk(own_left_edge(v_ref), right_halo(vf_ref), 3, 3, my).wait_send()

    return pl.pallas_call(
        body,
        out_shape=jax.ShapeDtypeStruct((B, Sl, D), jnp.float32),
        in_specs=[pl.BlockSpec(memory_space=pltpu.VMEM)] * 5,
        out_specs=pl.BlockSpec(memory_space=pltpu.VMEM),
        scratch_shapes=[
            pltpu.VMEM((B, H, Sf, Dh), jnp.float32),
            pltpu.VMEM((B, H, Sf, Dh), jnp.float32),
            pltpu.VMEM((B, Sl, Dq), jnp.float32),
            pltpu.SemaphoreType.DMA((4,)),
            pltpu.SemaphoreType.DMA((4,)),
        ],
        compiler_params=pltpu.CompilerParams(collective_id=0),
    )(x, Wq, Kt, Vt, Wo)


# device time: 16238 ns/iter; 1.4877x vs baseline; 1.4027x over previous
import jax
import jax.numpy as jnp
from jax import lax
from jax.experimental import pallas as pl
from jax.experimental.pallas import tpu as pltpu

N_DEV = 4
HALO = 128


def kernel(x, Wq, K_ext, V_ext, Wo):
    B, Sl, D = x.shape
    _, _, H, Dh = K_ext.shape
    Dq = Wq.shape[1]
    Sf = 2 * Sl
    CL = Sl
    CR = Sl + HALO

    Kt = jnp.transpose(K_ext, (0, 2, 3, 1))
    Vt = jnp.transpose(V_ext, (0, 2, 3, 1))

    def body(xs_ref, wqs_ref, kts_ref, vts_ref, wos_ref, outs_ref,
             kf_ref, vf_ref, q_ref, wo_bf_ref,
             send_sems, recv_sems):
        my = lax.axis_index("i")
        has_left = my > 0
        has_right = my < N_DEV - 1

        def mk(src, dst, s_idx, r_idx, dev):
            return pltpu.make_async_remote_copy(
                src_ref=src,
                dst_ref=dst,
                send_sem=send_sems.at[s_idx],
                recv_sem=recv_sems.at[r_idx],
                device_id=(dev,),
                device_id_type=pl.DeviceIdType.MESH,
            )

        def own_left(ref):
            return ref.at[:, :, :, pl.ds(0, HALO)]

        def own_right(ref):
            return ref.at[:, :, :, pl.ds(Sl - HALO, HALO)]

        def left_halo(ref):
            return ref.at[:, :, :, pl.ds(CL, HALO)]

        def right_halo(ref):
            return ref.at[:, :, :, pl.ds(CR, HALO)]

        barrier_sem = pltpu.get_barrier_semaphore()
        sig_l = jnp.where(has_left, my - 1, my)
        sig_r = jnp.where(has_right, my + 1, my)
        for tgt in (sig_l, sig_r):
            pl.semaphore_signal(
                barrier_sem, inc=1,
                device_id=(tgt,), device_id_type=pl.DeviceIdType.MESH,
            )
        pl.semaphore_wait(barrier_sem, 2)

        kf_ref[:, :, :, pl.ds(0, Sl)] = kts_ref[...].astype(jnp.bfloat16)

        @pl.when(has_right)
        def _():
            mk(own_right(kf_ref), left_halo(kf_ref), 0, 0, my + 1).start()

        @pl.when(has_left)
        def _():
            mk(own_left(kf_ref), right_halo(kf_ref), 2, 2, my - 1).start()

        vf_ref[:, :, :, pl.ds(0, Sl)] = vts_ref[...].astype(jnp.bfloat16)

        @pl.when(has_right)
        def _():
            mk(own_right(vf_ref), left_halo(vf_ref), 1, 1, my + 1).start()

        @pl.when(has_left)
        def _():
            mk(own_left(vf_ref), right_halo(vf_ref), 3, 3, my - 1).start()

        zero = jnp.zeros((B, H, Dh, HALO), jnp.bfloat16)

        @pl.when(jnp.logical_not(has_left))
        def _():
            left_halo(kf_ref)[...] = zero
            left_halo(vf_ref)[...] = zero

        @pl.when(jnp.logical_not(has_right))
        def _():
            right_halo(kf_ref)[...] = zero
            right_halo(vf_ref)[...] = zero

        wo_bf_ref[...] = wos_ref[...].astype(jnp.bfloat16)
        wq_bf = wqs_ref[...].astype(jnp.bfloat16)
        for b in range(B):
            q_ref[b] = (
                jnp.dot(
                    xs_ref[b].astype(jnp.bfloat16), wq_bf,
                    preferred_element_type=jnp.float32,
                ) * 0.125
            ).astype(jnp.bfloat16)

        i_c = lax.broadcasted_iota(jnp.int32, (Sl, Sl), 0)
        j_c = lax.broadcasted_iota(jnp.int32, (Sl, Sl), 1)
        mask_c = (
            (j_c >= i_c - HALO) & (j_c <= i_c + HALO)
        ).astype(jnp.float32)
        i_h = lax.broadcasted_iota(jnp.int32, (HALO, HALO), 0)
        j_h = lax.broadcasted_iota(jnp.int32, (HALO, HALO), 1)
        mask_l = ((j_h >= i_h) & has_left).astype(jnp.float32)
        mask_r = ((i_h >= j_h) & has_right).astype(jnp.float32)

        def nn_dot(a, bmat):
            return lax.dot_general(
                a, bmat, (((1,), (0,)), ((), ())),
                preferred_element_type=jnp.float32,
            )

        def nt_dot(a, bmat):
            return lax.dot_general(
                a, bmat, (((1,), (1,)), ((), ())),
                preferred_element_type=jnp.float32,
            )

        num = [[None] * H for _ in range(B)]
        den = [[None] * H for _ in range(B)]
        for b in range(B):
            for h in range(H):
                q_bh = q_ref[b, :, pl.ds(h * Dh, Dh)]
                w = jnp.exp(
                    nn_dot(q_bh, kf_ref[b, h, :, pl.ds(0, Sl)])
                ) * mask_c
                den[b][h] = jnp.sum(w, axis=1, keepdims=True)
                num[b][h] = nt_dot(
                    w.astype(jnp.bfloat16), vf_ref[b, h, :, pl.ds(0, Sl)]
                )

        @pl.when(has_left)
        def _():
            mk(own_left(kf_ref), left_halo(kf_ref), 0, 0, my).wait_recv()
            mk(own_left(vf_ref), left_halo(vf_ref), 1, 1, my).wait_recv()

        @pl.when(has_right)
        def _():
            mk(own_left(kf_ref), right_halo(kf_ref), 2, 2, my).wait_recv()
            mk(own_left(vf_ref), right_halo(vf_ref), 3, 3, my).wait_recv()

        for b in range(B):
            ctxs = []
            for h in range(H):
                q_top = q_ref[b, pl.ds(0, HALO), pl.ds(h * Dh, Dh)]
                q_bot = q_ref[b, pl.ds(HALO, HALO), pl.ds(h * Dh, Dh)]
                w_l = jnp.exp(
                    nn_dot(q_top, kf_ref[b, h, :, pl.ds(CL, HALO)])
                ) * mask_l
                w_r = jnp.exp(
                    nn_dot(q_bot, kf_ref[b, h, :, pl.ds(CR, HALO)])
                ) * mask_r
                d_t = den[b][h] + jnp.concatenate(
                    [
                        jnp.sum(w_l, axis=1, keepdims=True),
                        jnp.sum(w_r, axis=1, keepdims=True),
                    ],
                    axis=0,
                )
                n_t = num[b][h] + jnp.concatenate(
                    [
                        nt_dot(
                            w_l.astype(jnp.bfloat16),
                            vf_ref[b, h, :, pl.ds(CL, HALO)],
                        ),
                        nt_dot(
                            w_r.astype(jnp.bfloat16),
                            vf_ref[b, h, :, pl.ds(CR, HALO)],
                        ),
                    ],
                    axis=0,
                )
                ctxs.append((n_t / d_t).astype(jnp.bfloat16))
            outs_ref[b] = jnp.dot(
                jnp.concatenate(ctxs, axis=1), wo_bf_ref[...],
                preferred_element_type=jnp.float32,
            ).astype(jnp.bfloat16)

        @pl.when(has_right)
        def _():
            mk(own_right(kf_ref), left_halo(kf_ref), 0, 0, my).wait_send()
            mk(own_right(vf_ref), left_halo(vf_ref), 1, 1, my).wait_send()

        @pl.when(has_left)
        def _():
            mk(own_left(kf_ref), right_halo(kf_ref), 2, 2, my).wait_send()
            mk(own_left(vf_ref), right_halo(vf_ref), 3, 3, my).wait_send()

    return pl.pallas_call(
        body,
        out_shape=jax.ShapeDtypeStruct((B, Sl, D), jnp.bfloat16),
        in_specs=[pl.BlockSpec(memory_space=pltpu.VMEM)] * 5,
        out_specs=pl.BlockSpec(memory_space=pltpu.VMEM),
        scratch_shapes=[
            pltpu.VMEM((B, H, Dh, Sf), jnp.bfloat16),
            pltpu.VMEM((B, H, Dh, Sf), jnp.bfloat16),
            pltpu.VMEM((B, Sl, Dq), jnp.bfloat16),
            pltpu.VMEM((Dq, D), jnp.bfloat16),
            pltpu.SemaphoreType.DMA((4,)),
            pltpu.SemaphoreType.DMA((4,)),
        ],
        compiler_params=pltpu.CompilerParams(collective_id=0),
    )(x, Wq, Kt, Vt, Wo)
